# baseline (speedup 1.0000x reference)
import functools
import jax
import jax.numpy as jnp
from jax import lax
from jax.experimental import pallas as pl
from jax.experimental.pallas import tpu as pltpu
from jax.experimental.pallas import tpu_sc as plsc

_THRESH = 0.7
_MIN_KEPT = 100000

_N, _C, _H, _W = 8, 19, 512, 512
_HW = _H * _W
_NPIX = _N * _HW
_BLK = 2048
_NSTEP = _HW // _BLK


_BH = 8


def _stats_body(pred_ref, tgt_ref, p_ref, nll_ref):
    for n in range(_N):
        x = pred_ref[n]
        t = tgt_ref[n]
        m = x[0]
        for c in range(1, _C):
            m = jnp.maximum(m, x[c])
        s = jnp.zeros_like(m)
        tl = jnp.zeros_like(m)
        for c in range(_C):
            xc = x[c]
            s = s + jnp.exp(xc - m)
            tl = tl + jnp.where(t == c, xc, 0.0)
        p_ref[pl.ds(n * _BH, _BH), :] = jnp.exp(tl - m) / s
        nll_ref[pl.ds(n * _BH, _BH), :] = (m - tl) + jnp.log(s)


def _i32_const(v):
    v &= 0xFFFFFFFF
    if v >= 1 << 31:
        v -= 1 << 32
    return jnp.int32(v)


def _select_body(p_ref, nll_ref, out_ref):
    ch = 128
    nrows = _N * _H
    nch = nrows // ch
    kf = jnp.float32(_MIN_KEPT)

    prefix = jnp.int32(0)
    k_rem = kf
    for shift in range(28, -1, -4):
        nbin = 4 if shift == 28 else 16
        mask_ge = _i32_const(0xFFFFFFFF << shift)

        def chunk(i, accs, shift=shift, mask_ge=mask_ge, prefix=prefix,
                  nbin=nbin):
            sl = p_ref[pl.ds(i * ch, ch), :]
            bits = lax.bitcast_convert_type(sl, jnp.int32)
            y = bits & mask_ge
            out = []
            for b in range(nbin):
                tgt = prefix | lax.shift_left(jnp.int32(b), jnp.int32(shift))
                oh = jnp.where(y == tgt, 1.0, 0.0)
                a = accs[b]
                for j in range(ch // 8):
                    a = a + oh[j * 8:(j + 1) * 8, :]
                out.append(a)
            return tuple(out)

        accs = tuple(jnp.zeros((8, _W), jnp.float32) for _ in range(nbin))
        accs = lax.fori_loop(0, nch, chunk, accs)
        cnts = [jnp.sum(a) for a in accs]

        cumb = jnp.float32(0.0)
        sel_b = jnp.int32(nbin - 1)
        sel_cumb = jnp.float32(0.0)
        found = jnp.bool_(False)
        for b in range(nbin):
            hit = jnp.logical_and(jnp.logical_not(found),
                                  cumb + cnts[b] >= k_rem)
            sel_b = jnp.where(hit, jnp.int32(b), sel_b)
            sel_cumb = jnp.where(hit, cumb, sel_cumb)
            found = jnp.logical_or(found, hit)
            cumb = cumb + cnts[b]
        prefix = prefix | lax.shift_left(sel_b, jnp.int32(shift))
        k_rem = k_rem - sel_cumb

    thr = jnp.maximum(lax.bitcast_convert_type(prefix, jnp.float32),
                      jnp.float32(_THRESH))

    def red(i, carry):
        s_nll, s_cnt = carry
        pv = p_ref[pl.ds(i * ch, ch), :]
        nv = nll_ref[pl.ds(i * ch, ch), :]
        kept = pv <= thr
        s_nll = s_nll + jnp.sum(jnp.where(kept, nv, 0.0))
        s_cnt = s_cnt + jnp.sum(jnp.where(kept, 1.0, 0.0))
        return s_nll, s_cnt

    s_nll, s_cnt = lax.fori_loop(
        0, nch, red, (jnp.float32(0.0), jnp.float32(0.0)))
    out_ref[...] = (s_nll / jnp.maximum(s_cnt, 1.0)) * jnp.ones(
        (1, 1), jnp.float32)


_NT = 16
_NROWS = _N * _H
_ROWS_PT = _NROWS // _NT
_CHROWS = 32
_NCH = _ROWS_PT // _CHROWS
_L = 16
_SHIFTS = (23, 15, 7, 0)
_FBITS = (8, 8, 8, 7)


def _sc_select_body(p_hbm, thr_hbm, buf, lhist, allh, merged, tvec,
                    shared, sem):
    sid = lax.axis_index("s")
    row0 = sid * _ROWS_PT
    lane = lax.iota(jnp.int32, _L)
    lanebase = lane * 256
    ones = jnp.full((_L,), 1, jnp.int32)

    prefix = jnp.int32(0)
    k_rem = jnp.float32(_MIN_KEPT)

    for pi, (shift, fb) in enumerate(zip(_SHIFTS, _FBITS)):
        mask_above = _i32_const(0xFFFFFFFF << (shift + fb))
        fmask = jnp.int32((1 << fb) - 1)
        pref_hi = prefix & mask_above

        def zero(i, _):
            lhist[pl.ds(i * _L, _L)] = jnp.zeros((_L,), jnp.int32)
            return 0
        lax.fori_loop(0, (_L * 256) // _L, zero, 0)

        def chunk(c, carry, shift=shift, fmask=fmask,
                  mask_above=mask_above, pref_hi=pref_hi):
            pltpu.sync_copy(
                p_hbm.at[pl.ds(row0 + c * _CHROWS, _CHROWS), :], buf)

            def row(r, _):
                for cc in range(_W // _L):
                    v = buf[r, pl.ds(cc * _L, _L)]
                    bits = lax.bitcast_convert_type(v, jnp.int32)
                    ok = (bits & mask_above) == pref_hi
                    nib = (bits >> shift) & fmask
                    plsc.addupdate_scatter(
                        lhist, [lanebase + (nib ^ lane)], ones, mask=ok)
                return 0
            lax.fori_loop(0, _CHROWS, row, 0)
            return carry
        lax.fori_loop(0, _NCH, chunk, 0)

        def lmerge(k, _):
            b = lane + k * _L
            acc = jnp.zeros((_L,), jnp.int32)
            for l in range(_L):
                acc = acc + plsc.load_gather(lhist, [(b ^ l) + l * 256])
            merged[pl.ds(k * _L, _L)] = acc
            return 0
        lax.fori_loop(0, 256 // _L, lmerge, 0)

        pltpu.sync_copy(merged, shared.at[pi, sid])
        plsc.subcore_barrier()
        pltpu.sync_copy(shared.at[pi], allh)

        cum = jnp.float32(0.0)
        sel_b = jnp.int32((1 << fb) - 1)
        sel_cumb = jnp.float32(0.0)
        found = jnp.int32(0)
        for k in range((1 << fb) // _L):
            g = allh[0, pl.ds(k * _L, _L)]
            for l in range(1, _NT):
                g = g + allh[l, pl.ds(k * _L, _L)]
            gf = g.astype(jnp.float32)
            cumv = plsc.cumsum(gf)
            hit = jnp.where((cum + cumv >= k_rem) & (found == 0),
                            jnp.int32(1), jnp.int32(0))
            hcum = plsc.cumsum(hit)
            first = ((hit == 1) & (hcum == 1)).astype(jnp.float32)
            prevcum = cum + cumv - gf
            sel_cumb = sel_cumb + jnp.sum(prevcum * first)
            binid = (lane + k * _L).astype(jnp.float32)
            anyhit = jnp.sum(first)
            sel_b = jnp.where(anyhit > 0.0,
                              jnp.sum(binid * first).astype(jnp.int32),
                              sel_b)
            found = jnp.where(anyhit > 0.0, jnp.int32(1), found)
            cum = cum + jnp.sum(gf)
        prefix = prefix | lax.shift_left(sel_b, shift)
        k_rem = k_rem - sel_cumb

    thrbits = jnp.full((_L,), prefix, jnp.int32)
    thrf = jnp.maximum(lax.bitcast_convert_type(thrbits, jnp.float32),
                       jnp.full((_L,), _THRESH, jnp.float32))
    tvec[...] = thrf

    @pl.when(sid == 0)
    def _():
        pltpu.sync_copy(tvec, thr_hbm.at[0])


def _sc_select(p):
    mesh = plsc.VectorSubcoreMesh(core_axis_name="c",
                                  subcore_axis_name="s", num_cores=1)
    fn = pl.kernel(
        _sc_select_body, mesh=mesh,
        compiler_params=pltpu.CompilerParams(needs_layout_passes=False),
        out_type=jax.ShapeDtypeStruct((1, _L), jnp.float32),
        scratch_types=[
            pltpu.VMEM((_CHROWS, _W), jnp.float32),
            pltpu.VMEM((_L * 256,), jnp.int32),
            pltpu.VMEM((_NT, 256), jnp.int32),
            pltpu.VMEM((256,), jnp.int32),
            pltpu.VMEM((_L,), jnp.float32),
            pltpu.VMEM_SHARED((4, _NT, 256), jnp.int32),
            pltpu.SemaphoreType.DMA,
        ],
    )
    return fn(p)


def _final_body(p_ref, nll_ref, thr_ref, out_ref):
    ch = 128
    nch = (_N * _H) // ch
    thr = jnp.max(thr_ref[...])

    def red(i, carry):
        s_nll, s_cnt = carry
        pv = p_ref[pl.ds(i * ch, ch), :]
        nv = nll_ref[pl.ds(i * ch, ch), :]
        kept = pv <= thr
        s_nll = s_nll + jnp.sum(jnp.where(kept, nv, 0.0))
        s_cnt = s_cnt + jnp.sum(jnp.where(kept, 1.0, 0.0))
        return s_nll, s_cnt

    s_nll, s_cnt = lax.fori_loop(
        0, nch, red, (jnp.float32(0.0), jnp.float32(0.0)))
    out_ref[...] = (s_nll / jnp.maximum(s_cnt, 1.0)) * jnp.ones(
        (1, 1), jnp.float32)


def kernel(pred, target):
    p, nll = pl.pallas_call(
        _stats_body,
        grid=(_H // _BH,),
        in_specs=[
            pl.BlockSpec((_N, _C, _BH, _W), lambda i: (0, 0, i, 0)),
            pl.BlockSpec((_N, _BH, _W), lambda i: (0, i, 0)),
        ],
        out_specs=[
            pl.BlockSpec((_N * _BH, _W), lambda i: (i, 0)),
            pl.BlockSpec((_N * _BH, _W), lambda i: (i, 0)),
        ],
        out_shape=[
            jax.ShapeDtypeStruct((_N * _H, _W), jnp.float32),
            jax.ShapeDtypeStruct((_N * _H, _W), jnp.float32),
        ],
    )(pred, target)

    thr = _sc_select(p)

    loss = pl.pallas_call(
        _final_body,
        in_specs=[
            pl.BlockSpec((_N * _H, _W), lambda: (0, 0)),
            pl.BlockSpec((_N * _H, _W), lambda: (0, 0)),
            pl.BlockSpec((1, _L), lambda: (0, 0)),
        ],
        out_specs=pl.BlockSpec((1, 1), lambda: (0, 0)),
        out_shape=jax.ShapeDtypeStruct((1, 1), jnp.float32),
    )(p, nll, thr)
    return loss.reshape(())


# speedup vs baseline: 1.9542x; 1.9542x over previous
import functools
import jax
import jax.numpy as jnp
from jax import lax
from jax.experimental import pallas as pl
from jax.experimental.pallas import tpu as pltpu
from jax.experimental.pallas import tpu_sc as plsc

_THRESH = 0.7
_MIN_KEPT = 100000

_N, _C, _H, _W = 8, 19, 512, 512
_HW = _H * _W
_NPIX = _N * _HW
_BLK = 2048
_NSTEP = _HW // _BLK


_BH = 8


def _stats_body(pred_ref, tgt_ref, p_ref, nll_ref):
    for n in range(_N):
        x = pred_ref[n]
        t = tgt_ref[n]
        m = x[0]
        for c in range(1, _C):
            m = jnp.maximum(m, x[c])
        s = jnp.zeros_like(m)
        tl = jnp.zeros_like(m)
        for c in range(_C):
            xc = x[c]
            s = s + jnp.exp(xc - m)
            tl = tl + jnp.where(t == c, xc, 0.0)
        p_ref[pl.ds(n * _BH, _BH), :] = jnp.exp(tl - m) / s
        nll_ref[pl.ds(n * _BH, _BH), :] = (m - tl) + jnp.log(s)


def _i32_const(v):
    v &= 0xFFFFFFFF
    if v >= 1 << 31:
        v -= 1 << 32
    return jnp.int32(v)


def _select_body(p_ref, nll_ref, out_ref):
    ch = 128
    nrows = _N * _H
    nch = nrows // ch
    kf = jnp.float32(_MIN_KEPT)

    prefix = jnp.int32(0)
    k_rem = kf
    for shift in range(28, -1, -4):
        nbin = 4 if shift == 28 else 16
        mask_ge = _i32_const(0xFFFFFFFF << shift)

        def chunk(i, accs, shift=shift, mask_ge=mask_ge, prefix=prefix,
                  nbin=nbin):
            sl = p_ref[pl.ds(i * ch, ch), :]
            bits = lax.bitcast_convert_type(sl, jnp.int32)
            y = bits & mask_ge
            out = []
            for b in range(nbin):
                tgt = prefix | lax.shift_left(jnp.int32(b), jnp.int32(shift))
                oh = jnp.where(y == tgt, 1.0, 0.0)
                a = accs[b]
                for j in range(ch // 8):
                    a = a + oh[j * 8:(j + 1) * 8, :]
                out.append(a)
            return tuple(out)

        accs = tuple(jnp.zeros((8, _W), jnp.float32) for _ in range(nbin))
        accs = lax.fori_loop(0, nch, chunk, accs)
        cnts = [jnp.sum(a) for a in accs]

        cumb = jnp.float32(0.0)
        sel_b = jnp.int32(nbin - 1)
        sel_cumb = jnp.float32(0.0)
        found = jnp.bool_(False)
        for b in range(nbin):
            hit = jnp.logical_and(jnp.logical_not(found),
                                  cumb + cnts[b] >= k_rem)
            sel_b = jnp.where(hit, jnp.int32(b), sel_b)
            sel_cumb = jnp.where(hit, cumb, sel_cumb)
            found = jnp.logical_or(found, hit)
            cumb = cumb + cnts[b]
        prefix = prefix | lax.shift_left(sel_b, jnp.int32(shift))
        k_rem = k_rem - sel_cumb

    thr = jnp.maximum(lax.bitcast_convert_type(prefix, jnp.float32),
                      jnp.float32(_THRESH))

    def red(i, carry):
        s_nll, s_cnt = carry
        pv = p_ref[pl.ds(i * ch, ch), :]
        nv = nll_ref[pl.ds(i * ch, ch), :]
        kept = pv <= thr
        s_nll = s_nll + jnp.sum(jnp.where(kept, nv, 0.0))
        s_cnt = s_cnt + jnp.sum(jnp.where(kept, 1.0, 0.0))
        return s_nll, s_cnt

    s_nll, s_cnt = lax.fori_loop(
        0, nch, red, (jnp.float32(0.0), jnp.float32(0.0)))
    out_ref[...] = (s_nll / jnp.maximum(s_cnt, 1.0)) * jnp.ones(
        (1, 1), jnp.float32)


_NT = 16
_NROWS = _N * _H
_ROWS_PT = _NROWS // _NT
_CHROWS = 32
_NCH = _ROWS_PT // _CHROWS
_L = 16
_SHIFTS = (23, 15, 7, 0)
_FBITS = (8, 8, 8, 7)


def _sc_select_body(p_hbm, thr_hbm, buf, lhist, allh, merged, tvec,
                    shared, sem):
    sid = lax.axis_index("s")
    row0 = sid * _ROWS_PT
    lane = lax.iota(jnp.int32, _L)
    lanebase = lane * 256
    ones = jnp.full((_L,), 1, jnp.int32)

    prefix = jnp.int32(0)
    k_rem = jnp.float32(_MIN_KEPT)

    for pi, (shift, fb) in enumerate(zip(_SHIFTS, _FBITS)):
        mask_above = _i32_const(0xFFFFFFFF << (shift + fb))
        fmask = jnp.int32((1 << fb) - 1)
        pref_hi = prefix & mask_above

        def zero(i, _):
            lhist[pl.ds(i * _L, _L)] = jnp.zeros((_L,), jnp.int32)
            return 0
        lax.fori_loop(0, (_L * 256) // _L, zero, 0)

        def chunk(c, carry, shift=shift, fmask=fmask,
                  mask_above=mask_above, pref_hi=pref_hi):
            pltpu.sync_copy(
                p_hbm.at[pl.ds(row0 + c * _CHROWS, _CHROWS), :], buf)

            @plsc.parallel_loop(0, _CHROWS, 1, unroll=2)
            def row(r):
                for cc in range(_W // _L):
                    v = buf[r, pl.ds(cc * _L, _L)]
                    bits = lax.bitcast_convert_type(v, jnp.int32)
                    ok = (bits & mask_above) == pref_hi
                    nib = (bits >> shift) & fmask
                    plsc.addupdate_scatter(
                        lhist, [lanebase + (nib ^ lane)], ones, mask=ok)
            return carry
        lax.fori_loop(0, _NCH, chunk, 0)

        def lmerge(k, _):
            b = lane + k * _L
            acc = jnp.zeros((_L,), jnp.int32)
            for l in range(_L):
                acc = acc + plsc.load_gather(lhist, [(b ^ l) + l * 256])
            merged[pl.ds(k * _L, _L)] = acc
            return 0
        lax.fori_loop(0, 256 // _L, lmerge, 0)

        pltpu.sync_copy(merged, shared.at[pi, sid])
        plsc.subcore_barrier()
        pltpu.sync_copy(shared.at[pi], allh)

        cum = jnp.float32(0.0)
        sel_b = jnp.int32((1 << fb) - 1)
        sel_cumb = jnp.float32(0.0)
        found = jnp.int32(0)
        for k in range((1 << fb) // _L):
            g = allh[0, pl.ds(k * _L, _L)]
            for l in range(1, _NT):
                g = g + allh[l, pl.ds(k * _L, _L)]
            gf = g.astype(jnp.float32)
            cumv = plsc.cumsum(gf)
            hit = jnp.where((cum + cumv >= k_rem) & (found == 0),
                            jnp.int32(1), jnp.int32(0))
            hcum = plsc.cumsum(hit)
            first = ((hit == 1) & (hcum == 1)).astype(jnp.float32)
            prevcum = cum + cumv - gf
            sel_cumb = sel_cumb + jnp.sum(prevcum * first)
            binid = (lane + k * _L).astype(jnp.float32)
            anyhit = jnp.sum(first)
            sel_b = jnp.where(anyhit > 0.0,
                              jnp.sum(binid * first).astype(jnp.int32),
                              sel_b)
            found = jnp.where(anyhit > 0.0, jnp.int32(1), found)
            cum = cum + jnp.sum(gf)
        prefix = prefix | lax.shift_left(sel_b, shift)
        k_rem = k_rem - sel_cumb

    thrbits = jnp.full((_L,), prefix, jnp.int32)
    thrf = jnp.maximum(lax.bitcast_convert_type(thrbits, jnp.float32),
                       jnp.full((_L,), _THRESH, jnp.float32))
    tvec[...] = thrf

    @pl.when(sid == 0)
    def _():
        pltpu.sync_copy(tvec, thr_hbm.at[0])


def _sc_select(p):
    mesh = plsc.VectorSubcoreMesh(core_axis_name="c",
                                  subcore_axis_name="s", num_cores=1)
    fn = pl.kernel(
        _sc_select_body, mesh=mesh,
        compiler_params=pltpu.CompilerParams(needs_layout_passes=False),
        out_type=jax.ShapeDtypeStruct((1, _L), jnp.float32),
        scratch_types=[
            pltpu.VMEM((_CHROWS, _W), jnp.float32),
            pltpu.VMEM((_L * 256,), jnp.int32),
            pltpu.VMEM((_NT, 256), jnp.int32),
            pltpu.VMEM((256,), jnp.int32),
            pltpu.VMEM((_L,), jnp.float32),
            pltpu.VMEM_SHARED((4, _NT, 256), jnp.int32),
            pltpu.SemaphoreType.DMA,
        ],
    )
    return fn(p)


def _final_body(p_ref, nll_ref, thr_ref, out_ref):
    ch = 128
    nch = (_N * _H) // ch
    thr = jnp.max(thr_ref[...])

    def red(i, carry):
        s_nll, s_cnt = carry
        pv = p_ref[pl.ds(i * ch, ch), :]
        nv = nll_ref[pl.ds(i * ch, ch), :]
        kept = pv <= thr
        s_nll = s_nll + jnp.sum(jnp.where(kept, nv, 0.0))
        s_cnt = s_cnt + jnp.sum(jnp.where(kept, 1.0, 0.0))
        return s_nll, s_cnt

    s_nll, s_cnt = lax.fori_loop(
        0, nch, red, (jnp.float32(0.0), jnp.float32(0.0)))
    out_ref[...] = (s_nll / jnp.maximum(s_cnt, 1.0)) * jnp.ones(
        (1, 1), jnp.float32)


def kernel(pred, target):
    p, nll = pl.pallas_call(
        _stats_body,
        grid=(_H // _BH,),
        in_specs=[
            pl.BlockSpec((_N, _C, _BH, _W), lambda i: (0, 0, i, 0)),
            pl.BlockSpec((_N, _BH, _W), lambda i: (0, i, 0)),
        ],
        out_specs=[
            pl.BlockSpec((_N * _BH, _W), lambda i: (i, 0)),
            pl.BlockSpec((_N * _BH, _W), lambda i: (i, 0)),
        ],
        out_shape=[
            jax.ShapeDtypeStruct((_N * _H, _W), jnp.float32),
            jax.ShapeDtypeStruct((_N * _H, _W), jnp.float32),
        ],
    )(pred, target)

    thr = _sc_select(p)

    loss = pl.pallas_call(
        _final_body,
        in_specs=[
            pl.BlockSpec((_N * _H, _W), lambda: (0, 0)),
            pl.BlockSpec((_N * _H, _W), lambda: (0, 0)),
            pl.BlockSpec((1, _L), lambda: (0, 0)),
        ],
        out_specs=pl.BlockSpec((1, 1), lambda: (0, 0)),
        out_shape=jax.ShapeDtypeStruct((1, 1), jnp.float32),
    )(p, nll, thr)
    return loss.reshape(())
